# baseline (speedup 1.0000x reference)
import functools

import jax
import jax.numpy as jnp
from jax.experimental import pallas as pl
from jax.experimental.pallas import tpu as pltpu

_K = 16
_NEG = -1e30


def _pass1_kernel(n_real, emb_s, emb_full, rnrm_row, x_blk, lin_w, thr_ref, g_ref):
    dot = jax.lax.dot_general(emb_s[...], emb_full[...],
                              (((1,), (1,)), ((), ())),
                              preferred_element_type=jnp.float32)
    score = dot * rnrm_row[...]
    tcol = jax.lax.broadcasted_iota(jnp.int32, score.shape, 1)
    score = jnp.where(tcol < n_real, score, _NEG)

    v = score
    v16 = None
    v17 = None
    for k in range(_K + 1):
        m = jnp.max(v, axis=1, keepdims=True)
        if k == _K - 1:
            v16 = m
        if k == _K:
            v17 = m
        if k < _K:
            v = jnp.where(v >= m, _NEG, v)
    thr_ref[...] = jnp.where(v17 < v16, 0.5 * (v16 + v17), v16)

    for b in range(x_blk.shape[0]):
        g_ref[b, :, :] = jax.lax.dot_general(x_blk[b], lin_w[...],
                                             (((1,), (1,)), ((), ())),
                                             preferred_element_type=jnp.float32)


def _pass2_kernel(n_real, tt, sc,
                  emb_t, emb_s, rnrm_t, thr_row, g_t, g_s,
                  ai_col, aei_col, aj_row, aej_row,
                  glb_row, scale_row, beta_row, ow_col, ob,
                  out_ref, m_s, d_s, z_s):
    ti = pl.program_id(0)
    sj = pl.program_id(1)
    ns = pl.num_programs(1)
    nb = g_t.shape[0]

    @pl.when(sj == 0)
    def _init():
        m_s[...] = jnp.full(m_s.shape, _NEG, jnp.float32)
        d_s[...] = jnp.zeros(d_s.shape, jnp.float32)
        z_s[...] = jnp.zeros(z_s.shape, jnp.float32)

    dot = jax.lax.dot_general(emb_t[...], emb_s[...],
                              (((1,), (1,)), ((), ())),
                              preferred_element_type=jnp.float32)
    score = dot * rnrm_t[...]
    t_col = jax.lax.broadcasted_iota(jnp.int32, (tt, 1), 0) + ti * tt
    s_row = jax.lax.broadcasted_iota(jnp.int32, (1, sc), 1) + sj * sc
    eye = t_col == s_row
    mask = (s_row < n_real) & ((score >= thr_row[...]) | eye)

    for b in range(nb):
        p = (jax.lax.dot_general(g_t[b], ai_col[...], (((1,), (0,)), ((), ())),
                                 preferred_element_type=jnp.float32)
             + jax.lax.dot_general(emb_t[...], aei_col[...],
                                   (((1,), (0,)), ((), ())),
                                   preferred_element_type=jnp.float32))
        q = (jax.lax.dot_general(aj_row[...], g_s[b], (((1,), (1,)), ((), ())),
                                 preferred_element_type=jnp.float32)
             + jax.lax.dot_general(aej_row[...], emb_s[...],
                                   (((1,), (1,)), ((), ())),
                                   preferred_element_type=jnp.float32))
        lg = p + q
        lg = jnp.where(lg >= 0, lg, 0.2 * lg)
        lg = jnp.where(mask, lg, _NEG)

        cm = jnp.max(lg, axis=1, keepdims=True)
        m_old = m_s[b]
        m_new = jnp.maximum(m_old, cm)
        corr = jnp.exp(m_old - m_new)
        m_s[b] = m_new
        a = jnp.where(mask, jnp.exp(lg - m_new), 0.0)
        d_s[b] = d_s[b] * corr + jnp.sum(a, axis=1, keepdims=True)
        z_s[b] = z_s[b] * corr + jax.lax.dot_general(
            a.astype(jnp.bfloat16), g_s[b].astype(jnp.bfloat16),
            (((1,), (0,)), ((), ())),
            preferred_element_type=jnp.float32)

    @pl.when(sj == ns - 1)
    def _fin():
        for b in range(nb):
            z = z_s[b] / (d_s[b] + 1e-16) + glb_row[...]
            sv = z * emb_t[...]
            sv = sv * scale_row[...] + beta_row[...]
            sv = jnp.maximum(sv, 0.0)
            o = jax.lax.dot_general(sv, ow_col[...], (((1,), (0,)), ((), ())),
                                    preferred_element_type=jnp.float32)
            out_ref[:, b:b + 1] = o + ob[...]


def kernel(x, emb_table, lin_W, att_i, att_j, att_em_i, att_em_j,
           gl_bias, bn_gamma, bn_beta, out_W, out_b):
    B, N, L = x.shape
    D = emb_table.shape[1]

    SC = 1024
    SB = 256
    TT = 256
    npad = -(-N // SC) * SC
    pad = npad - N

    emb_p = jnp.pad(emb_table, ((0, pad), (0, 0)))
    x_p = jnp.pad(x, ((0, 0), (0, pad), (0, 0)))
    nrm2 = jnp.sum(emb_p * emb_p, axis=1)
    rnrm = jnp.where(nrm2 > 0, 1.0 / jnp.sqrt(nrm2), 0.0)
    rnrm_row = rnrm.reshape(1, npad)
    rnrm_col = rnrm.reshape(npad, 1)

    thr_col, g = pl.pallas_call(
        functools.partial(_pass1_kernel, N),
        grid=(npad // SB,),
        in_specs=[
            pl.BlockSpec((SB, D), lambda i: (i, 0)),
            pl.BlockSpec((npad, D), lambda i: (0, 0)),
            pl.BlockSpec((1, npad), lambda i: (0, 0)),
            pl.BlockSpec((B, SB, L), lambda i: (0, i, 0)),
            pl.BlockSpec((D, L), lambda i: (0, 0)),
        ],
        out_specs=[
            pl.BlockSpec((SB, 1), lambda i: (i, 0)),
            pl.BlockSpec((B, SB, D), lambda i: (0, i, 0)),
        ],
        out_shape=[
            jax.ShapeDtypeStruct((npad, 1), jnp.float32),
            jax.ShapeDtypeStruct((B, npad, D), jnp.float32),
        ],
    )(emb_p, emb_p, rnrm_row, x_p, lin_W)

    thr_row = thr_col.reshape(1, npad)
    scale_row = (bn_gamma / jnp.sqrt(1.0 + 1e-5)).reshape(1, D)

    out_pad = pl.pallas_call(
        functools.partial(_pass2_kernel, N, TT, SC),
        grid=(npad // TT, npad // SC),
        in_specs=[
            pl.BlockSpec((TT, D), lambda i, j: (i, 0)),
            pl.BlockSpec((SC, D), lambda i, j: (j, 0)),
            pl.BlockSpec((TT, 1), lambda i, j: (i, 0)),
            pl.BlockSpec((1, SC), lambda i, j: (0, j)),
            pl.BlockSpec((B, TT, D), lambda i, j: (0, i, 0)),
            pl.BlockSpec((B, SC, D), lambda i, j: (0, j, 0)),
            pl.BlockSpec((D, 1), lambda i, j: (0, 0)),
            pl.BlockSpec((D, 1), lambda i, j: (0, 0)),
            pl.BlockSpec((1, D), lambda i, j: (0, 0)),
            pl.BlockSpec((1, D), lambda i, j: (0, 0)),
            pl.BlockSpec((1, D), lambda i, j: (0, 0)),
            pl.BlockSpec((1, D), lambda i, j: (0, 0)),
            pl.BlockSpec((1, D), lambda i, j: (0, 0)),
            pl.BlockSpec((D, 1), lambda i, j: (0, 0)),
            pl.BlockSpec((1, 1), lambda i, j: (0, 0)),
        ],
        out_specs=pl.BlockSpec((TT, B), lambda i, j: (i, 0)),
        out_shape=jax.ShapeDtypeStruct((npad, B), jnp.float32),
        scratch_shapes=[
            pltpu.VMEM((B, TT, 1), jnp.float32),
            pltpu.VMEM((B, TT, 1), jnp.float32),
            pltpu.VMEM((B, TT, D), jnp.float32),
        ],
    )(emb_p, emb_p, rnrm_col, thr_row, g, g,
      att_i.reshape(D, 1), att_em_i.reshape(D, 1),
      att_j.reshape(1, D), att_em_j.reshape(1, D),
      gl_bias.reshape(1, D), scale_row, bn_beta.reshape(1, D),
      out_W.reshape(D, 1), out_b.reshape(1, 1))

    return out_pad[:N, :].T


# speedup vs baseline: 1.0417x; 1.0417x over previous
import functools

import jax
import jax.numpy as jnp
from jax.experimental import pallas as pl
from jax.experimental.pallas import tpu as pltpu

_K = 16
_NEG = -1e30


def _pass1_kernel(n_real, emb_s, emb_full, rnrm_row, x_blk, lin_w, thr_ref, g_ref):
    dot = jax.lax.dot_general(emb_s[...], emb_full[...],
                              (((1,), (1,)), ((), ())),
                              preferred_element_type=jnp.float32)
    score = dot * rnrm_row[...]
    tcol = jax.lax.broadcasted_iota(jnp.int32, score.shape, 1)
    score = jnp.where(tcol < n_real, score, _NEG)

    v = score
    v16 = None
    v17 = None
    for k in range(_K + 1):
        m = jnp.max(v, axis=1, keepdims=True)
        if k == _K - 1:
            v16 = m
        if k == _K:
            v17 = m
        if k < _K:
            v = jnp.where(v >= m, _NEG, v)
    thr_ref[...] = jnp.where(v17 < v16, 0.5 * (v16 + v17), v16)

    for b in range(x_blk.shape[0]):
        g_ref[b, :, :] = jax.lax.dot_general(x_blk[b], lin_w[...],
                                             (((1,), (1,)), ((), ())),
                                             preferred_element_type=jnp.float32)


def _pass2_kernel(n_real, tt, sc,
                  emb_t, emb_s, rnrm_t, thr_row, g_t, g_s,
                  ai_col, aei_col, aj_row, aej_row,
                  glb_row, scale_row, beta_row, ow_col, ob,
                  out_ref, m_s, d_s, z_s):
    ti = pl.program_id(0)
    sj = pl.program_id(1)
    ns = pl.num_programs(1)
    nb = g_t.shape[0]

    @pl.when(sj == 0)
    def _init():
        m_s[...] = jnp.full(m_s.shape, -1e6, jnp.float32)
        d_s[...] = jnp.zeros(d_s.shape, jnp.float32)
        z_s[...] = jnp.zeros(z_s.shape, jnp.float32)

    dot = jax.lax.dot_general(emb_t[...], emb_s[...],
                              (((1,), (1,)), ((), ())),
                              preferred_element_type=jnp.float32)
    score = dot * rnrm_t[...]
    t_col = jax.lax.broadcasted_iota(jnp.int32, (tt, 1), 0) + ti * tt
    s_row = jax.lax.broadcasted_iota(jnp.int32, (1, sc), 1) + sj * sc
    eye = t_col == s_row
    mask = (s_row < n_real) & ((score >= thr_row[...]) | eye)
    maskadd = jnp.where(mask, 0.0, _NEG)

    for b in range(nb):
        p = (jax.lax.dot_general(g_t[b], ai_col[...], (((1,), (0,)), ((), ())),
                                 preferred_element_type=jnp.float32)
             + jax.lax.dot_general(emb_t[...], aei_col[...],
                                   (((1,), (0,)), ((), ())),
                                   preferred_element_type=jnp.float32))
        q = (jax.lax.dot_general(aj_row[...], g_s[b], (((1,), (1,)), ((), ())),
                                 preferred_element_type=jnp.float32)
             + jax.lax.dot_general(aej_row[...], emb_s[...],
                                   (((1,), (1,)), ((), ())),
                                   preferred_element_type=jnp.float32))
        lg = p + q
        lg = jnp.maximum(lg, 0.2 * lg) + maskadd

        cm = jnp.max(lg, axis=1, keepdims=True)
        m_old = m_s[b]
        m_new = jnp.maximum(m_old, cm)
        corr = jnp.exp(m_old - m_new)
        m_s[b] = m_new
        a = jnp.exp(lg - m_new)
        d_s[b] = d_s[b] * corr + jnp.sum(a, axis=1, keepdims=True)
        z_s[b] = z_s[b] * corr + jax.lax.dot_general(
            a, g_s[b], (((1,), (0,)), ((), ())),
            preferred_element_type=jnp.float32)

    @pl.when(sj == ns - 1)
    def _fin():
        for b in range(nb):
            z = z_s[b] / (d_s[b] + 1e-16) + glb_row[...]
            sv = z * emb_t[...]
            sv = sv * scale_row[...] + beta_row[...]
            sv = jnp.maximum(sv, 0.0)
            o = jax.lax.dot_general(sv, ow_col[...], (((1,), (0,)), ((), ())),
                                    preferred_element_type=jnp.float32)
            out_ref[:, b:b + 1] = o + ob[...]


def kernel(x, emb_table, lin_W, att_i, att_j, att_em_i, att_em_j,
           gl_bias, bn_gamma, bn_beta, out_W, out_b):
    B, N, L = x.shape
    D = emb_table.shape[1]

    SC = 1024
    SB = 256
    TT = 256
    npad = -(-N // SC) * SC
    pad = npad - N

    emb_p = jnp.pad(emb_table, ((0, pad), (0, 0)))
    x_p = jnp.pad(x, ((0, 0), (0, pad), (0, 0)))
    nrm2 = jnp.sum(emb_p * emb_p, axis=1)
    rnrm = jnp.where(nrm2 > 0, 1.0 / jnp.sqrt(nrm2), 0.0)
    rnrm_row = rnrm.reshape(1, npad)
    rnrm_col = rnrm.reshape(npad, 1)

    thr_col, g = pl.pallas_call(
        functools.partial(_pass1_kernel, N),
        grid=(npad // SB,),
        in_specs=[
            pl.BlockSpec((SB, D), lambda i: (i, 0)),
            pl.BlockSpec((npad, D), lambda i: (0, 0)),
            pl.BlockSpec((1, npad), lambda i: (0, 0)),
            pl.BlockSpec((B, SB, L), lambda i: (0, i, 0)),
            pl.BlockSpec((D, L), lambda i: (0, 0)),
        ],
        out_specs=[
            pl.BlockSpec((SB, 1), lambda i: (i, 0)),
            pl.BlockSpec((B, SB, D), lambda i: (0, i, 0)),
        ],
        out_shape=[
            jax.ShapeDtypeStruct((npad, 1), jnp.float32),
            jax.ShapeDtypeStruct((B, npad, D), jnp.float32),
        ],
    )(emb_p, emb_p, rnrm_row, x_p, lin_W)

    thr_row = thr_col.reshape(1, npad)
    scale_row = (bn_gamma / jnp.sqrt(1.0 + 1e-5)).reshape(1, D)

    out_pad = pl.pallas_call(
        functools.partial(_pass2_kernel, N, TT, SC),
        grid=(npad // TT, npad // SC),
        in_specs=[
            pl.BlockSpec((TT, D), lambda i, j: (i, 0)),
            pl.BlockSpec((SC, D), lambda i, j: (j, 0)),
            pl.BlockSpec((TT, 1), lambda i, j: (i, 0)),
            pl.BlockSpec((1, SC), lambda i, j: (0, j)),
            pl.BlockSpec((B, TT, D), lambda i, j: (0, i, 0)),
            pl.BlockSpec((B, SC, D), lambda i, j: (0, j, 0)),
            pl.BlockSpec((D, 1), lambda i, j: (0, 0)),
            pl.BlockSpec((D, 1), lambda i, j: (0, 0)),
            pl.BlockSpec((1, D), lambda i, j: (0, 0)),
            pl.BlockSpec((1, D), lambda i, j: (0, 0)),
            pl.BlockSpec((1, D), lambda i, j: (0, 0)),
            pl.BlockSpec((1, D), lambda i, j: (0, 0)),
            pl.BlockSpec((1, D), lambda i, j: (0, 0)),
            pl.BlockSpec((D, 1), lambda i, j: (0, 0)),
            pl.BlockSpec((1, 1), lambda i, j: (0, 0)),
        ],
        out_specs=pl.BlockSpec((TT, B), lambda i, j: (i, 0)),
        out_shape=jax.ShapeDtypeStruct((npad, B), jnp.float32),
        scratch_shapes=[
            pltpu.VMEM((B, TT, 1), jnp.float32),
            pltpu.VMEM((B, TT, 1), jnp.float32),
            pltpu.VMEM((B, TT, D), jnp.float32),
        ],
    )(emb_p, emb_p, rnrm_col, thr_row, g, g,
      att_i.reshape(D, 1), att_em_i.reshape(D, 1),
      att_j.reshape(1, D), att_em_j.reshape(1, D),
      gl_bias.reshape(1, D), scale_row, bn_beta.reshape(1, D),
      out_W.reshape(D, 1), out_b.reshape(1, 1))

    return out_pad[:N, :].T
